# baseline (speedup 1.0000x reference)
import functools

import jax
import jax.numpy as jnp
from jax import lax
from jax.experimental import pallas as pl
from jax.experimental.pallas import tpu as pltpu
from jax.experimental.pallas import tpu_sc as plsc

B = 16384
D = 64
N = 100000
NC = 2
NS = 16
NW = NC * NS
FPW = D // NW
CH = 2048
NCH = B // CH
LANES = 16


@functools.partial(
    pl.kernel,
    out_type=jax.ShapeDtypeStruct((D, B), jnp.float32),
    mesh=plsc.VectorSubcoreMesh(core_axis_name="c", subcore_axis_name="s"),
    compiler_params=pltpu.CompilerParams(needs_layout_passes=False),
    scratch_types=[
        pltpu.VMEM((N,), jnp.float32),
        pltpu.VMEM((B,), jnp.float32),
        pltpu.VMEM((CH,), jnp.int32),
        pltpu.VMEM((CH,), jnp.int32),
        pltpu.VMEM((CH,), jnp.int32),
        pltpu.VMEM((CH,), jnp.int32),
        pltpu.SemaphoreType.DMA,
        pltpu.SemaphoreType.DMA,
        pltpu.SemaphoreType.DMA,
        pltpu.SemaphoreType.DMA,
        pltpu.SemaphoreType.DMA,
        pltpu.SemaphoreType.DMA,
        pltpu.SemaphoreType.DMA,
    ],
)
def _plane_affine(xt_hbm, idx_hbm, st_hbm, ht_hbm, out_hbm,
                  plane_v, row_v, idx0_v, idx1_v, idx2_v, idx3_v,
                  sem_p, sem_x, sem_i0, sem_i1, sem_i2, sem_i3, sem_o):
    wid = lax.axis_index("s") * NC + lax.axis_index("c")

    idxb = (idx0_v, idx1_v, idx2_v, idx3_v)
    isem = (sem_i0, sem_i1, sem_i2, sem_i3)

    def fetch_idx(c, buf):
        pltpu.async_copy(idx_hbm.at[pl.ds(c * CH, CH)], idxb[buf], isem[buf])

    def wait_idx(buf):
        pltpu.make_async_copy(
            idx_hbm.at[pl.ds(0, CH)], idxb[buf], isem[buf]).wait()

    def sweep(idx_ref, cbase, mul):
        @plsc.parallel_loop(0, CH, LANES, unroll=16)
        def body(i):
            iv = idx_ref[pl.ds(i, LANES)]
            g = plsc.load_gather(plane_v, [iv])
            s = pl.ds(cbase + i, LANES)
            if mul:
                row_v[s] = row_v[s] * g
            else:
                row_v[s] = row_v[s] + g

    def pass_(mul):
        def quad(q, carry):
            for u in range(4):
                c = 4 * q + u
                wait_idx(u)
                sweep(idxb[u], c * CH, mul)
                fetch_idx((c + 4) % NCH, u)
            return carry

        lax.fori_loop(0, NCH // 4, quad, 0)

    for u in range(4):
        fetch_idx(u, u)

    def feat(f, carry):
        j = wid * FPW + f
        pltpu.async_copy(xt_hbm.at[j], row_v, sem_x)
        pltpu.async_copy(st_hbm.at[j], plane_v, sem_p)
        pltpu.make_async_copy(st_hbm.at[j], plane_v, sem_p).wait()
        pltpu.make_async_copy(xt_hbm.at[j], row_v, sem_x).wait()
        pass_(mul=True)
        pltpu.async_copy(ht_hbm.at[j], plane_v, sem_p)
        pltpu.make_async_copy(ht_hbm.at[j], plane_v, sem_p).wait()
        pass_(mul=False)
        pltpu.async_copy(row_v, out_hbm.at[j], sem_o)
        pltpu.make_async_copy(row_v, out_hbm.at[j], sem_o).wait()
        return carry

    lax.fori_loop(0, FPW, feat, 0)

    for u in range(4):
        wait_idx(u)


def kernel(x, batch_idx, scale_weight, shift_weight):
    idx = jnp.asarray(batch_idx, jnp.int32)
    out_t = _plane_affine(x.T, idx, scale_weight.T, shift_weight.T)
    return out_t.T


# speedup vs baseline: 1.0187x; 1.0187x over previous
import functools

import jax
import jax.numpy as jnp
from jax import lax
from jax.experimental import pallas as pl
from jax.experimental.pallas import tpu as pltpu
from jax.experimental.pallas import tpu_sc as plsc

B = 16384
D = 64
N = 100000
NC = 2
NS = 16
NW = NC * NS
FPW = D // NW
CH = 2048
NCH = B // CH
LANES = 16


@functools.partial(
    pl.kernel,
    out_type=jax.ShapeDtypeStruct((D, B), jnp.float32),
    mesh=plsc.VectorSubcoreMesh(core_axis_name="c", subcore_axis_name="s"),
    compiler_params=pltpu.CompilerParams(needs_layout_passes=False),
    scratch_types=[
        pltpu.VMEM((N,), jnp.float32),
        pltpu.VMEM((B,), jnp.float32),
        pltpu.VMEM((CH,), jnp.int32),
        pltpu.VMEM((CH,), jnp.int32),
        pltpu.VMEM((CH,), jnp.int32),
        pltpu.VMEM((CH,), jnp.int32),
        pltpu.SemaphoreType.DMA,
        pltpu.SemaphoreType.DMA,
        pltpu.SemaphoreType.DMA,
        pltpu.SemaphoreType.DMA,
        pltpu.SemaphoreType.DMA,
        pltpu.SemaphoreType.DMA,
        pltpu.SemaphoreType.DMA,
    ],
)
def _plane_affine(xt_hbm, idx_hbm, st_hbm, ht_hbm, out_hbm,
                  plane_v, row_v, idx0_v, idx1_v, idx2_v, idx3_v,
                  sem_p, sem_x, sem_i0, sem_i1, sem_i2, sem_i3, sem_o):
    wid = lax.axis_index("s") * NC + lax.axis_index("c")

    idxb = (idx0_v, idx1_v, idx2_v, idx3_v)
    isem = (sem_i0, sem_i1, sem_i2, sem_i3)

    def fetch_idx(c, buf):
        pltpu.async_copy(idx_hbm.at[pl.ds(c * CH, CH)], idxb[buf], isem[buf])

    def wait_idx(buf):
        pltpu.make_async_copy(
            idx_hbm.at[pl.ds(0, CH)], idxb[buf], isem[buf]).wait()

    def sweep(idx_ref, cbase, mul):
        @plsc.parallel_loop(0, CH, LANES, unroll=8)
        def body(i):
            iv = idx_ref[pl.ds(i, LANES)]
            g = plsc.load_gather(plane_v, [iv])
            s = pl.ds(cbase + i, LANES)
            if mul:
                row_v[s] = row_v[s] * g
            else:
                row_v[s] = row_v[s] + g

    def pass_(mul):
        def quad(q, carry):
            for u in range(4):
                c = 4 * q + u
                wait_idx(u)
                sweep(idxb[u], c * CH, mul)
                fetch_idx((c + 4) % NCH, u)
            return carry

        lax.fori_loop(0, NCH // 4, quad, 0)

    for u in range(4):
        fetch_idx(u, u)

    def feat(f, carry):
        j = wid * FPW + f
        pltpu.async_copy(xt_hbm.at[j], row_v, sem_x)
        pltpu.async_copy(st_hbm.at[j], plane_v, sem_p)
        pltpu.make_async_copy(st_hbm.at[j], plane_v, sem_p).wait()
        pltpu.make_async_copy(xt_hbm.at[j], row_v, sem_x).wait()
        pass_(mul=True)
        pltpu.async_copy(ht_hbm.at[j], plane_v, sem_p)
        pltpu.make_async_copy(ht_hbm.at[j], plane_v, sem_p).wait()
        pass_(mul=False)
        pltpu.async_copy(row_v, out_hbm.at[j], sem_o)
        pltpu.make_async_copy(row_v, out_hbm.at[j], sem_o).wait()
        return carry

    lax.fori_loop(0, FPW, feat, 0)

    for u in range(4):
        wait_idx(u)


def kernel(x, batch_idx, scale_weight, shift_weight):
    idx = jnp.asarray(batch_idx, jnp.int32)
    out_t = _plane_affine(x.T, idx, scale_weight.T, shift_weight.T)
    return out_t.T
